# Initial kernel scaffold; baseline (speedup 1.0000x reference)
import jax
import jax.numpy as jnp
from jax.experimental import pallas as pl


def kernel(image, point_cloud, img_points, pn1_w, pn1_b, pn2_w, pn2_b, pn3_w, pn3_b, c1_w, c1_b, c2_w, c2_b, c3_w, c3_b, c4_w, c4_b, c5_w, c5_b, proj_large_w, proj_large_b, proj_medium_w, proj_medium_b, proj_small_w, proj_small_b, pred_large_w, pred_large_b, pred_medium_w, pred_medium_b, pred_small_w, pred_small_b):
    raise NotImplementedError("write your pallas kernel here")


# speedup vs baseline: 1.1213x; 1.1213x over previous
import functools

import jax
import jax.numpy as jnp
from jax.experimental import pallas as pl
from jax.experimental.pallas import tpu as pltpu

_VMEM = 48 * 1024 * 1024


def _rup(x, m):
    return ((x + m - 1) // m) * m


def _mm_kernel(a_ref, w_ref, b_ref, o_ref, *, act):
    acc = jnp.dot(a_ref[...], w_ref[...], preferred_element_type=jnp.float32)
    acc = acc + b_ref[...]
    if act == "leaky":
        acc = jnp.where(acc > 0, acc, 0.1 * acc)
    elif act == "relu":
        acc = jnp.maximum(acc, 0.0)
    o_ref[...] = acc.astype(o_ref.dtype)


def _mm(a, w, b, act="none", out_dtype=jnp.bfloat16):
    M, K = a.shape
    N = w.shape[1]
    a = a.astype(jnp.bfloat16)
    w = w.astype(jnp.bfloat16)
    b = b.astype(jnp.float32)

    Kp = _rup(K, 16)
    Np = _rup(N, 128)
    tn = 256 if Np % 256 == 0 else 128

    tm = None
    for t in (512, 256, 128, 64, 32, 16, 8):
        if M % t == 0:
            tm = t
            break
    if tm is None:
        tm = min(512, _rup(M, 8))
    Mp = _rup(M, tm)

    if (Mp, Kp) != (M, K):
        a = jnp.pad(a, ((0, Mp - M), (0, Kp - K)))
    if (Kp, Np) != (K, N):
        w = jnp.pad(w, ((0, Kp - K), (0, Np - N)))
    if Np != N:
        b = jnp.pad(b, (0, Np - N))
    b2 = b.reshape(1, Np)

    out = pl.pallas_call(
        functools.partial(_mm_kernel, act=act),
        out_shape=jax.ShapeDtypeStruct((Mp, Np), out_dtype),
        grid=(Mp // tm, Np // tn),
        in_specs=[
            pl.BlockSpec((tm, Kp), lambda i, j: (i, 0)),
            pl.BlockSpec((Kp, tn), lambda i, j: (0, j)),
            pl.BlockSpec((1, tn), lambda i, j: (0, j)),
        ],
        out_specs=pl.BlockSpec((tm, tn), lambda i, j: (i, j)),
        compiler_params=pltpu.CompilerParams(
            dimension_semantics=("parallel", "parallel"),
            vmem_limit_bytes=_VMEM),
    )(a, w, b2)
    return out[:M, :N]


def _conv3x3_s2(x_nhwc, w, b):
    B, H, W, C = x_nhwc.shape
    xp = jnp.pad(x_nhwc, ((0, 0), (1, 1), (1, 1), (0, 0)))
    Ho, Wo = H // 2, W // 2
    cols = []
    for dh in range(3):
        for dw in range(3):
            cols.append(xp[:, dh:dh + 2 * Ho:2, dw:dw + 2 * Wo:2, :])
    patches = jnp.concatenate(cols, axis=-1)
    Cout = w.shape[1]
    out = _mm(patches.reshape(B * Ho * Wo, 9 * C), w, b, act="leaky")
    return out.reshape(B, Ho, Wo, Cout)


def _pn_kernel(x_ref, w1_ref, b1_ref, w2_ref, b2_ref, w3_ref, b3_ref, o_ref):
    x = x_ref[0]
    h = jnp.dot(x, w1_ref[...], preferred_element_type=jnp.float32) + b1_ref[...]
    h = jnp.maximum(h, 0.0)
    h = jnp.dot(h.astype(jnp.bfloat16), w2_ref[...],
                preferred_element_type=jnp.float32) + b2_ref[...]
    h = jnp.maximum(h, 0.0)
    h = jnp.dot(h.astype(jnp.bfloat16), w3_ref[...],
                preferred_element_type=jnp.float32) + b3_ref[...]
    m = jnp.max(h, axis=0, keepdims=True)

    @pl.when(pl.program_id(1) == 0)
    def _():
        o_ref[0] = m

    @pl.when(pl.program_id(1) > 0)
    def _():
        o_ref[0] = jnp.maximum(o_ref[0], m)


def _pointnet(point_cloud, w1, b1, w2, b2, w3, b3):
    B, N, Cin = point_cloud.shape
    F1, F2, F3 = w1.shape[1], w2.shape[1], w3.shape[1]
    Cp = _rup(Cin, 8)

    tile = 1000
    assert N % tile == 0

    pts = point_cloud.astype(jnp.bfloat16)
    if Cp != Cin:
        pts = jnp.pad(pts, ((0, 0), (0, 0), (0, Cp - Cin)))
        w1 = jnp.pad(w1, ((0, Cp - Cin), (0, 0)))

    out = pl.pallas_call(
        _pn_kernel,
        out_shape=jax.ShapeDtypeStruct((B, 1, F3), jnp.float32),
        grid=(B, N // tile),
        in_specs=[
            pl.BlockSpec((1, tile, Cp), lambda b_, c: (b_, c, 0)),
            pl.BlockSpec((Cp, F1), lambda b_, c: (0, 0)),
            pl.BlockSpec((1, F1), lambda b_, c: (0, 0)),
            pl.BlockSpec((F1, F2), lambda b_, c: (0, 0)),
            pl.BlockSpec((1, F2), lambda b_, c: (0, 0)),
            pl.BlockSpec((F2, F3), lambda b_, c: (0, 0)),
            pl.BlockSpec((1, F3), lambda b_, c: (0, 0)),
        ],
        out_specs=pl.BlockSpec((1, 1, F3), lambda b_, c: (b_, 0, 0)),
        compiler_params=pltpu.CompilerParams(
            dimension_semantics=("parallel", "arbitrary"),
            vmem_limit_bytes=_VMEM),
    )(pts,
      w1.astype(jnp.bfloat16), b1.reshape(1, F1).astype(jnp.float32),
      w2.astype(jnp.bfloat16), b2.reshape(1, F2).astype(jnp.float32),
      w3.astype(jnp.bfloat16), b3.reshape(1, F3).astype(jnp.float32))
    return out.reshape(B, F3)


def _head_kernel(f_ref, m_ref, l_ref, w_ref, b_ref, o_ref):
    fused = f_ref[0].astype(jnp.float32) + m_ref[0] * l_ref[0]
    out = jnp.dot(fused.astype(jnp.bfloat16), w_ref[...],
                  preferred_element_type=jnp.float32) + b_ref[...]
    o_ref[0] = out


def _head(feat_nhwc, mask_bhw1, lid_bc, w, b):
    B, Hs, Ws, C = feat_nhwc.shape
    HW = Hs * Ws
    n_out = w.shape[1]
    Np = _rup(n_out, 128)
    tr = min(HW, 3200)
    assert HW % tr == 0

    feat = feat_nhwc.reshape(B, HW, C)
    mask = mask_bhw1.astype(jnp.float32)
    lid = lid_bc.reshape(B, 1, C).astype(jnp.float32)
    wp = jnp.pad(w.astype(jnp.bfloat16), ((0, 0), (0, Np - n_out)))
    bp = jnp.pad(b.astype(jnp.float32), (0, Np - n_out)).reshape(1, Np)

    out = pl.pallas_call(
        _head_kernel,
        out_shape=jax.ShapeDtypeStruct((B, HW, Np), jnp.float32),
        grid=(B, HW // tr),
        in_specs=[
            pl.BlockSpec((1, tr, C), lambda bi, r: (bi, r, 0)),
            pl.BlockSpec((1, tr, 1), lambda bi, r: (bi, r, 0)),
            pl.BlockSpec((1, 1, C), lambda bi, r: (bi, 0, 0)),
            pl.BlockSpec((C, Np), lambda bi, r: (0, 0)),
            pl.BlockSpec((1, Np), lambda bi, r: (0, 0)),
        ],
        out_specs=pl.BlockSpec((1, tr, Np), lambda bi, r: (bi, r, 0)),
        compiler_params=pltpu.CompilerParams(
            dimension_semantics=("parallel", "parallel"),
            vmem_limit_bytes=_VMEM),
    )(feat, mask, lid, wp, bp)

    out = out[:, :, :n_out].reshape(B, Hs, Ws, n_out)
    return jnp.transpose(out, (0, 3, 1, 2))


def _occupancy_masks(img_points, H, W):
    B, N, _ = img_points.shape
    x = img_points[..., 0]
    y = img_points[..., 1]
    cx = jnp.clip(jnp.floor(x * 80 / W), 0, 79).astype(jnp.int32)
    cy = jnp.clip(jnp.floor(y * 80 / H), 0, 79).astype(jnp.int32)
    b_idx = jnp.broadcast_to(jnp.arange(B)[:, None], (B, N))
    m80 = jnp.zeros((B, 80, 80), jnp.float32).at[b_idx, cy, cx].max(1.0)
    m40 = jnp.max(m80.reshape(B, 40, 2, 40, 2), axis=(2, 4))
    m20 = jnp.max(m40.reshape(B, 20, 2, 20, 2), axis=(2, 4))
    return m80, m40, m20


def kernel(image, point_cloud, img_points,
           pn1_w, pn1_b, pn2_w, pn2_b, pn3_w, pn3_b,
           c1_w, c1_b, c2_w, c2_b, c3_w, c3_b, c4_w, c4_b, c5_w, c5_b,
           proj_large_w, proj_large_b, proj_medium_w, proj_medium_b,
           proj_small_w, proj_small_b,
           pred_large_w, pred_large_b, pred_medium_w, pred_medium_b,
           pred_small_w, pred_small_b):
    B, _, H, W = image.shape

    x = jnp.transpose(image, (0, 2, 3, 1)).astype(jnp.bfloat16)
    x = _conv3x3_s2(x, c1_w, c1_b)
    x = _conv3x3_s2(x, c2_w, c2_b)
    small = _conv3x3_s2(x, c3_w, c3_b)
    medium = _conv3x3_s2(small, c4_w, c4_b)
    large = _conv3x3_s2(medium, c5_w, c5_b)

    lidar = _pointnet(point_cloud, pn1_w, pn1_b, pn2_w, pn2_b, pn3_w, pn3_b)

    w_all = jnp.concatenate([proj_large_w, proj_medium_w, proj_small_w], axis=1)
    b_all = jnp.concatenate([proj_large_b, proj_medium_b, proj_small_b], axis=0)
    lid_all = _mm(lidar, w_all, b_all, act="none", out_dtype=jnp.float32)

    m80, m40, m20 = _occupancy_masks(img_points, H, W)

    out_large = _head(large, m20.reshape(B, 400, 1), lid_all[:, :512],
                      pred_large_w, pred_large_b)
    out_medium = _head(medium, m40.reshape(B, 1600, 1), lid_all[:, 512:768],
                       pred_medium_w, pred_medium_b)
    out_small = _head(small, m80.reshape(B, 6400, 1), lid_all[:, 768:896],
                      pred_small_w, pred_small_b)
    return [out_large, out_medium, out_small]


# speedup vs baseline: 1.2505x; 1.1152x over previous
import functools

import jax
import jax.numpy as jnp
from jax.experimental import pallas as pl
from jax.experimental.pallas import tpu as pltpu

_VMEM = 48 * 1024 * 1024


def _rup(x, m):
    return ((x + m - 1) // m) * m


def _mm_kernel(a_ref, w_ref, b_ref, o_ref, *, act):
    acc = jnp.dot(a_ref[...], w_ref[...], preferred_element_type=jnp.float32)
    acc = acc + b_ref[...]
    if act == "leaky":
        acc = jnp.where(acc > 0, acc, 0.1 * acc)
    elif act == "relu":
        acc = jnp.maximum(acc, 0.0)
    o_ref[...] = acc.astype(o_ref.dtype)


def _mm(a, w, b, act="none", out_dtype=jnp.bfloat16):
    M, K = a.shape
    N = w.shape[1]
    a = a.astype(jnp.bfloat16)
    w = w.astype(jnp.bfloat16)
    b = b.astype(jnp.float32)

    Kp = _rup(K, 16)
    Np = _rup(N, 128)
    tn = 256 if Np % 256 == 0 else 128

    tm = None
    for t in (512, 256, 128, 64, 32, 16, 8):
        if M % t == 0:
            tm = t
            break
    if tm is None:
        tm = min(512, _rup(M, 8))
    Mp = _rup(M, tm)

    if (Mp, Kp) != (M, K):
        a = jnp.pad(a, ((0, Mp - M), (0, Kp - K)))
    if (Kp, Np) != (K, N):
        w = jnp.pad(w, ((0, Kp - K), (0, Np - N)))
    if Np != N:
        b = jnp.pad(b, (0, Np - N))
    b2 = b.reshape(1, Np)

    out = pl.pallas_call(
        functools.partial(_mm_kernel, act=act),
        out_shape=jax.ShapeDtypeStruct((Mp, Np), out_dtype),
        grid=(Mp // tm, Np // tn),
        in_specs=[
            pl.BlockSpec((tm, Kp), lambda i, j: (i, 0)),
            pl.BlockSpec((Kp, tn), lambda i, j: (0, j)),
            pl.BlockSpec((1, tn), lambda i, j: (0, j)),
        ],
        out_specs=pl.BlockSpec((tm, tn), lambda i, j: (i, j)),
        compiler_params=pltpu.CompilerParams(
            dimension_semantics=("parallel", "parallel"),
            vmem_limit_bytes=_VMEM),
    )(a, w, b2)
    return out[:M, :N]


def _conv3x3_s2(x_nhwc, w, b):
    B, H, W, C = x_nhwc.shape
    xp = jnp.pad(x_nhwc, ((0, 0), (1, 1), (1, 1), (0, 0)))
    Hh, Wh = (H + 2) // 2, (W + 2) // 2
    y = xp.reshape(B, Hh, 2, Wh, 2, C).transpose(0, 1, 3, 2, 4, 5)
    y = y.reshape(B, Hh, Wh, 4 * C)
    Ho, Wo = H // 2, W // 2
    cols = []
    for dh in range(3):
        for dw in range(3):
            a, dr = dh // 2, dh % 2
            c0, dc = dw // 2, dw % 2
            g = (dr * 2 + dc) * C
            cols.append(y[:, a:a + Ho, c0:c0 + Wo, g:g + C])
    patches = jnp.concatenate(cols, axis=-1)
    Cout = w.shape[1]
    out = _mm(patches.reshape(B * Ho * Wo, 9 * C), w, b, act="leaky")
    return out.reshape(B, Ho, Wo, Cout)


def _pn_kernel(x_ref, w1_ref, b1_ref, w2_ref, b2_ref, w3_ref, b3_ref, o_ref):
    x = x_ref[0]
    h = jnp.dot(x, w1_ref[...], preferred_element_type=jnp.float32) + b1_ref[...]
    h = jnp.maximum(h, 0.0)
    h = jnp.dot(h.astype(jnp.bfloat16), w2_ref[...],
                preferred_element_type=jnp.float32) + b2_ref[...]
    h = jnp.maximum(h, 0.0)
    h = jnp.dot(h.astype(jnp.bfloat16), w3_ref[...],
                preferred_element_type=jnp.float32) + b3_ref[...]
    m = jnp.max(h, axis=0, keepdims=True)

    @pl.when(pl.program_id(1) == 0)
    def _():
        o_ref[0] = m

    @pl.when(pl.program_id(1) > 0)
    def _():
        o_ref[0] = jnp.maximum(o_ref[0], m)


def _pointnet(point_cloud, w1, b1, w2, b2, w3, b3):
    B, N, Cin = point_cloud.shape
    F1, F2, F3 = w1.shape[1], w2.shape[1], w3.shape[1]
    Cp = _rup(Cin, 8)

    tile = 1000
    assert N % tile == 0

    pts = point_cloud.astype(jnp.bfloat16)
    if Cp != Cin:
        pts = jnp.pad(pts, ((0, 0), (0, 0), (0, Cp - Cin)))
        w1 = jnp.pad(w1, ((0, Cp - Cin), (0, 0)))

    out = pl.pallas_call(
        _pn_kernel,
        out_shape=jax.ShapeDtypeStruct((B, 1, F3), jnp.float32),
        grid=(B, N // tile),
        in_specs=[
            pl.BlockSpec((1, tile, Cp), lambda b_, c: (b_, c, 0)),
            pl.BlockSpec((Cp, F1), lambda b_, c: (0, 0)),
            pl.BlockSpec((1, F1), lambda b_, c: (0, 0)),
            pl.BlockSpec((F1, F2), lambda b_, c: (0, 0)),
            pl.BlockSpec((1, F2), lambda b_, c: (0, 0)),
            pl.BlockSpec((F2, F3), lambda b_, c: (0, 0)),
            pl.BlockSpec((1, F3), lambda b_, c: (0, 0)),
        ],
        out_specs=pl.BlockSpec((1, 1, F3), lambda b_, c: (b_, 0, 0)),
        compiler_params=pltpu.CompilerParams(
            dimension_semantics=("parallel", "arbitrary"),
            vmem_limit_bytes=_VMEM),
    )(pts,
      w1.astype(jnp.bfloat16), b1.reshape(1, F1).astype(jnp.float32),
      w2.astype(jnp.bfloat16), b2.reshape(1, F2).astype(jnp.float32),
      w3.astype(jnp.bfloat16), b3.reshape(1, F3).astype(jnp.float32))
    return out.reshape(B, F3)


def _head_kernel(f_ref, m_ref, l_ref, w_ref, b_ref, o_ref):
    fused = f_ref[0].astype(jnp.float32) + m_ref[0] * l_ref[0]
    out = jnp.dot(fused.astype(jnp.bfloat16), w_ref[...],
                  preferred_element_type=jnp.float32) + b_ref[...]
    o_ref[0] = out


def _head(feat_nhwc, mask_bhw1, lid_bc, w, b):
    B, Hs, Ws, C = feat_nhwc.shape
    HW = Hs * Ws
    n_out = w.shape[1]
    Np = _rup(n_out, 128)
    tr = min(HW, 3200)
    assert HW % tr == 0

    feat = feat_nhwc.reshape(B, HW, C)
    mask = mask_bhw1.astype(jnp.float32)
    lid = lid_bc.reshape(B, 1, C).astype(jnp.float32)
    wp = jnp.pad(w.astype(jnp.bfloat16), ((0, 0), (0, Np - n_out)))
    bp = jnp.pad(b.astype(jnp.float32), (0, Np - n_out)).reshape(1, Np)

    out = pl.pallas_call(
        _head_kernel,
        out_shape=jax.ShapeDtypeStruct((B, HW, Np), jnp.float32),
        grid=(B, HW // tr),
        in_specs=[
            pl.BlockSpec((1, tr, C), lambda bi, r: (bi, r, 0)),
            pl.BlockSpec((1, tr, 1), lambda bi, r: (bi, r, 0)),
            pl.BlockSpec((1, 1, C), lambda bi, r: (bi, 0, 0)),
            pl.BlockSpec((C, Np), lambda bi, r: (0, 0)),
            pl.BlockSpec((1, Np), lambda bi, r: (0, 0)),
        ],
        out_specs=pl.BlockSpec((1, tr, Np), lambda bi, r: (bi, r, 0)),
        compiler_params=pltpu.CompilerParams(
            dimension_semantics=("parallel", "parallel"),
            vmem_limit_bytes=_VMEM),
    )(feat, mask, lid, wp, bp)

    out = out[:, :, :n_out].reshape(B, Hs, Ws, n_out)
    return jnp.transpose(out, (0, 3, 1, 2))


def _occupancy_masks(img_points, H, W):
    B, N, _ = img_points.shape
    x = img_points[..., 0]
    y = img_points[..., 1]
    cx = jnp.clip(jnp.floor(x * 80 / W), 0, 79).astype(jnp.int32)
    cy = jnp.clip(jnp.floor(y * 80 / H), 0, 79).astype(jnp.int32)
    b_idx = jnp.broadcast_to(jnp.arange(B)[:, None], (B, N))
    m80 = jnp.zeros((B, 80, 80), jnp.float32).at[b_idx, cy, cx].max(1.0)
    m40 = jnp.max(m80.reshape(B, 40, 2, 40, 2), axis=(2, 4))
    m20 = jnp.max(m40.reshape(B, 20, 2, 20, 2), axis=(2, 4))
    return m80, m40, m20


def kernel(image, point_cloud, img_points,
           pn1_w, pn1_b, pn2_w, pn2_b, pn3_w, pn3_b,
           c1_w, c1_b, c2_w, c2_b, c3_w, c3_b, c4_w, c4_b, c5_w, c5_b,
           proj_large_w, proj_large_b, proj_medium_w, proj_medium_b,
           proj_small_w, proj_small_b,
           pred_large_w, pred_large_b, pred_medium_w, pred_medium_b,
           pred_small_w, pred_small_b):
    B, _, H, W = image.shape

    x = jnp.transpose(image, (0, 2, 3, 1)).astype(jnp.bfloat16)
    x = _conv3x3_s2(x, c1_w, c1_b)
    x = _conv3x3_s2(x, c2_w, c2_b)
    small = _conv3x3_s2(x, c3_w, c3_b)
    medium = _conv3x3_s2(small, c4_w, c4_b)
    large = _conv3x3_s2(medium, c5_w, c5_b)

    lidar = _pointnet(point_cloud, pn1_w, pn1_b, pn2_w, pn2_b, pn3_w, pn3_b)

    w_all = jnp.concatenate([proj_large_w, proj_medium_w, proj_small_w], axis=1)
    b_all = jnp.concatenate([proj_large_b, proj_medium_b, proj_small_b], axis=0)
    lid_all = _mm(lidar, w_all, b_all, act="none", out_dtype=jnp.float32)

    m80, m40, m20 = _occupancy_masks(img_points, H, W)

    out_large = _head(large, m20.reshape(B, 400, 1), lid_all[:, :512],
                      pred_large_w, pred_large_b)
    out_medium = _head(medium, m40.reshape(B, 1600, 1), lid_all[:, 512:768],
                       pred_medium_w, pred_medium_b)
    out_small = _head(small, m80.reshape(B, 6400, 1), lid_all[:, 768:896],
                      pred_small_w, pred_small_b)
    return [out_large, out_medium, out_small]
